# Initial kernel scaffold; baseline (speedup 1.0000x reference)
import jax
import jax.numpy as jnp
from jax.experimental import pallas as pl


def kernel(node_features, edge_index, edge_features, edge_types, W_node, b_node, g_node, be_node, W_edge, b_edge, g_edge, be_edge, et_emb, Wl, We, a_src, a_dst, a_e, w_pool, W_pool, b_pool, W_v1, b_v1, W_v2, b_v2, W_a1, b_a1, W_a2, b_a2):
    raise NotImplementedError("write your pallas kernel here")


# speedup vs baseline: 36.4884x; 36.4884x over previous
import functools

import jax
import jax.numpy as jnp
from jax import lax
from jax.experimental import pallas as pl
from jax.experimental.pallas import tpu as pltpu
from jax.experimental.pallas import tpu_sc as plsc

N = 50000
E = 800000
H = 64
HEADS = 8
DH = 8
L = 4
BN = 2000
BE = 4000
C = 400
NTILE = 16
EPT = E // NTILE
NCH = EPT // C
DR0 = 3128
DR1 = N - 15 * DR0

_f32 = jnp.float32


def _ln(x, g, b):
    m = jnp.mean(x, axis=-1, keepdims=True)
    d = x - m
    v = jnp.mean(d * d, axis=-1, keepdims=True)
    return d / jnp.sqrt(v + 1e-5) * g + b


def _node_prep_body(nf, Wn, bn, gn, ben, Wl0, us0, ud0,
                    x_o, h0_o, h1_o, as_o, ad_o):
    xb = jnp.dot(nf[...], Wn[...], preferred_element_type=_f32) + bn[...]
    xb = jax.nn.relu(_ln(xb, gn[...], ben[...]))
    x_o[...] = xb
    h = jnp.dot(xb, Wl0[...], preferred_element_type=_f32)
    h0_o[...] = h[:, :32]
    h1_o[...] = h[:, 32:]
    as_o[...] = jnp.dot(xb, us0[...], preferred_element_type=_f32)
    ad_o[...] = jnp.dot(xb, ud0[...], preferred_element_type=_f32)


def _node_prep(nf, Wn, bn, gn, ben, Wl0, us0, ud0):
    nb = N // BN
    full = lambda shape: pl.BlockSpec(shape, lambda i: (0,) * len(shape))
    row = lambda w: pl.BlockSpec((BN, w), lambda i: (i, 0))
    return pl.pallas_call(
        _node_prep_body,
        grid=(nb,),
        in_specs=[row(24), full((24, H)), full((1, H)), full((1, H)), full((1, H)),
                  full((H, H)), full((H, 8)), full((H, 8))],
        out_specs=[row(H), row(32), row(32), row(8), row(8)],
        out_shape=[jax.ShapeDtypeStruct((N, H), _f32),
                   jax.ShapeDtypeStruct((N, 32), _f32),
                   jax.ShapeDtypeStruct((N, 32), _f32),
                   jax.ShapeDtypeStruct((N, 8), _f32),
                   jax.ShapeDtypeStruct((N, 8), _f32)],
    )(nf, Wn, bn, gn, ben, Wl0, us0, ud0)


def _edge_body(ef, et, We_, be_, ge_, bee, etemb, ve,
               ae0_o, ae1_o, ae2_o, ae3_o):
    tcol = et[...]
    hot = (tcol == lax.broadcasted_iota(jnp.int32, (1, 8), 1)).astype(_f32)
    emb = jnp.dot(hot, etemb[...], preferred_element_type=_f32)
    ee = jnp.dot(ef[...], We_[...], preferred_element_type=_f32) + be_[...] + emb
    ee = jax.nn.relu(_ln(ee, ge_[...], bee[...]))
    ae = jnp.dot(ee, ve[...], preferred_element_type=_f32)
    ae0_o[...] = ae[:, 0:8]
    ae1_o[...] = ae[:, 8:16]
    ae2_o[...] = ae[:, 16:24]
    ae3_o[...] = ae[:, 24:32]


def _edge_prep(ef, et2, We_, be_, ge_, bee, etemb, ve):
    nb = E // BE
    full = lambda shape: pl.BlockSpec(shape, lambda i: (0,) * len(shape))
    row = lambda w: pl.BlockSpec((BE, w), lambda i: (i, 0))
    outs = pl.pallas_call(
        _edge_body,
        grid=(nb,),
        in_specs=[row(8), row(1), full((8, H)), full((1, H)), full((1, H)),
                  full((1, H)), full((8, H)), full((H, 32))],
        out_specs=[row(8)] * 4,
        out_shape=[jax.ShapeDtypeStruct((E, 8), _f32)] * 4,
    )(ef, et2, We_, be_, ge_, bee, etemb, ve)
    return [o.reshape(E // 2, 16) for o in outs]


_SC_MESH = None


def _sc_mesh():
    global _SC_MESH
    if _SC_MESH is None:
        _SC_MESH = plsc.VectorSubcoreMesh(core_axis_name="c", subcore_axis_name="s")
    return _SC_MESH


NR = 32
RW = 1564
PAD = 1600


def _sc_edge_body(src_hbm, dst_hbm, asrc_t, adst_t, ae_t, h0, h1, zacc,
                  sb_hbm, nc_hbm,
                  out0, out1,
                  src_v, dst_v, sbuf, dbuf, aebuf, exbuf, hbuf, acc,
                  sb_s, nc_s):
    cid = lax.axis_index("c")
    sid = lax.axis_index("s")
    pltpu.sync_copy(sb_hbm, sb_s)
    pltpu.sync_copy(nc_hbm, nc_s)

    def _scal(ref, i):
        return jnp.max(plsc.load_gather(ref, [jnp.zeros((16,), jnp.int32) + i]))

    iot = lax.iota(jnp.int32, 16)
    lane7 = iot & 7
    half8 = iot >> 3
    colh = 4 * cid + half8

    for r in range(2):
        rid = 2 * sid + r
        base0 = _scal(sb_s, rid)
        nch = _scal(nc_s, rid)
        nodebase = RW * rid
        pltpu.sync_copy(zacc, acc)

        def chunk(ci, _):
            base = pl.multiple_of(base0 + ci * C, 16)
            pltpu.sync_copy(src_hbm.at[pl.ds(base, C)], src_v)
            pltpu.sync_copy(dst_hbm.at[pl.ds(base, C)], dst_v)
            pltpu.sync_copy(asrc_t.at[src_v], sbuf)
            pltpu.sync_copy(adst_t.at[dst_v], dbuf)
            pltpu.sync_copy(ae_t.at[pl.ds(pl.multiple_of(base // 2, 8), C // 2)],
                            aebuf)

            @pl.when(cid == 0)
            def _():
                pltpu.sync_copy(h0.at[src_v], hbuf)

            @pl.when(cid == 1)
            def _():
                pltpu.sync_copy(h1.at[src_v], hbuf)

            def pair(j, _):
                a = 2 * j
                rows2 = a + half8
                g1 = plsc.load_gather(sbuf, [rows2, lane7])
                g2 = plsc.load_gather(dbuf, [rows2, lane7])
                lg = g1 + g2 + aebuf[j]
                lg = jnp.where(lg > 0, lg, 0.2 * lg)
                ex = jnp.exp(lg)
                exbuf[j] = ex
                rowj = jnp.zeros((16,), jnp.int32) + j
                for edge in range(2):
                    rv = a + edge + jnp.zeros((16,), jnp.int32)
                    dste = plsc.load_gather(dst_v, [rv])
                    loc = dste - nodebase
                    okv = (loc >= 0) & (loc < RW)
                    rowe = jnp.where(okv, loc, RW)
                    ev = plsc.load_gather(exbuf, [rowj, 8 * edge + 4 * cid + (iot & 3)])
                    plsc.addupdate_scatter(acc, [rowe, 32 + (iot & 3)], ev,
                                           mask=iot < 4)
                    for half in range(2):
                        eb = plsc.load_gather(exbuf,
                                              [rowj, 8 * edge + 2 * half + colh])
                        vals = plsc.load_gather(hbuf, [rv, 16 * half + iot]) * eb
                        plsc.addupdate_scatter(acc, [rowe, 16 * half + iot], vals)
                return 0

            lax.fori_loop(0, C // 2, pair, 0)
            return 0

        lax.fori_loop(0, nch, chunk, 0)
        off = pl.multiple_of(nodebase, 4)

        @pl.when(cid == 0)
        def _():
            pltpu.sync_copy(acc.at[pl.ds(0, RW)], out0.at[pl.ds(off, RW)])

        @pl.when(cid == 1)
        def _():
            pltpu.sync_copy(acc.at[pl.ds(0, RW)], out1.at[pl.ds(off, RW)])


def _sc_edge_pass(src, dst, asrc_t, adst_t, ae_t, h0, h1, zacc, sb, nc):
    k = functools.partial(
        pl.kernel,
        out_type=[jax.ShapeDtypeStruct((NR * RW, 36), _f32),
                  jax.ShapeDtypeStruct((NR * RW, 36), _f32)],
        mesh=_sc_mesh(),
        compiler_params=pltpu.CompilerParams(needs_layout_passes=False,
                                             use_tc_tiling_on_sc=False),
        scratch_types=[
            pltpu.VMEM((C,), jnp.int32),
            pltpu.VMEM((C,), jnp.int32),
            pltpu.VMEM((C, 8), _f32),
            pltpu.VMEM((C, 8), _f32),
            pltpu.VMEM((C // 2, 16), _f32),
            pltpu.VMEM((C // 2, 16), _f32),
            pltpu.VMEM((C, 32), _f32),
            pltpu.VMEM((RW + 4, 36), _f32),
            pltpu.VMEM((40,), jnp.int32),
            pltpu.VMEM((40,), jnp.int32),
        ],
    )(_sc_edge_body)
    return k(src, dst, asrc_t, adst_t, ae_t, h0, h1, zacc, sb, nc)


def _combine_body_mid(a0, a1, x, Wln, usn, udn,
                      x_o, h0_o, h1_o, as_o, ad_o):
    xn = _mix(a0[...], a1[...], x[...])
    x_o[...] = xn
    h = jnp.dot(xn, Wln[...], preferred_element_type=_f32)
    h0_o[...] = h[:, :32]
    h1_o[...] = h[:, 32:]
    as_o[...] = jnp.dot(xn, usn[...], preferred_element_type=_f32)
    ad_o[...] = jnp.dot(xn, udn[...], preferred_element_type=_f32)


def _mix(a0, a1, x):
    den = jnp.concatenate([a0[:, 32:36], a1[:, 32:36]], axis=1) + 1e-16
    krep = (lax.broadcasted_iota(jnp.int32, (8, H), 1) // 8
            == lax.broadcasted_iota(jnp.int32, (8, H), 0)).astype(_f32)
    den_rep = jnp.dot(den, krep, preferred_element_type=_f32)
    num = jnp.concatenate([a0[:, :32], a1[:, :32]], axis=1)
    o = num / den_rep
    o = jnp.where(o > 0, o, jnp.exp(o) - 1.0)
    return o + x


def _combine_body_last(a0, a1, x, x_o):
    x_o[...] = _mix(a0[...], a1[...], x[...])


def _combine(a0, a1, x, Wln=None, usn=None, udn=None):
    nb = N // BN
    full = lambda shape: pl.BlockSpec(shape, lambda i: (0,) * len(shape))
    row = lambda w: pl.BlockSpec((BN, w), lambda i: (i, 0))
    if Wln is None:
        return pl.pallas_call(
            _combine_body_last,
            grid=(nb,),
            in_specs=[row(36), row(36), row(H)],
            out_specs=row(H),
            out_shape=jax.ShapeDtypeStruct((N, H), _f32),
        )(a0, a1, x)
    return pl.pallas_call(
        _combine_body_mid,
        grid=(nb,),
        in_specs=[row(36), row(36), row(H),
                  full((H, H)), full((H, 8)), full((H, 8))],
        out_specs=[row(H), row(32), row(32), row(8), row(8)],
        out_shape=[jax.ShapeDtypeStruct((N, H), _f32),
                   jax.ShapeDtypeStruct((N, 32), _f32),
                   jax.ShapeDtypeStruct((N, 32), _f32),
                   jax.ShapeDtypeStruct((N, 8), _f32),
                   jax.ShapeDtypeStruct((N, 8), _f32)],
    )(a0, a1, x, Wln, usn, udn)


def _pool_body(x, wp, Wp, bp, Wv1, bv1, Wv2, bv2, Wa1, ba1, Wa2, ba2,
               q_o, m_s, s_s, v_s):
    i = pl.program_id(0)
    nb = pl.num_programs(0)

    @pl.when(i == 0)
    def _():
        m_s[...] = jnp.full((1, 1), -1e30, _f32)
        s_s[...] = jnp.zeros((1, 1), _f32)
        v_s[...] = jnp.zeros((1, H), _f32)

    xb = x[...]
    sb = jnp.dot(xb, wp[...], preferred_element_type=_f32)
    bm = jnp.max(sb, axis=(0, 1), keepdims=True)
    m_old = m_s[...]
    m_new = jnp.maximum(m_old, bm)
    e = jnp.exp(sb - m_new)
    scale = jnp.exp(m_old - m_new)
    s_s[...] = s_s[...] * scale + jnp.sum(e, axis=(0, 1), keepdims=True)
    contrib = lax.dot_general(e, xb, (((0,), (0,)), ((), ())),
                              preferred_element_type=_f32)
    v_s[...] = v_s[...] * scale + contrib
    m_s[...] = m_new

    @pl.when(i == nb - 1)
    def _():
        pooled = v_s[...] / s_s[...]
        p = jax.nn.relu(jnp.dot(pooled, Wp[...], preferred_element_type=_f32)
                        + bp[...])
        hv = jax.nn.relu(jnp.dot(p, Wv1[...], preferred_element_type=_f32)
                         + bv1[...])
        value = jnp.dot(hv, Wv2[...], preferred_element_type=_f32) + bv2[...]
        ha = jax.nn.relu(jnp.dot(p, Wa1[...], preferred_element_type=_f32)
                         + ba1[...])
        adv = jnp.dot(ha, Wa2[...], preferred_element_type=_f32) + ba2[...]
        mean_adv = jnp.mean(adv, axis=(0, 1), keepdims=True)
        q_o[...] = value + adv - mean_adv


def _pool_head(x, wp, Wp, bp, Wv1, bv1, Wv2, bv2, Wa1, ba1, Wa2, ba2):
    nb = N // BN
    A = 4672
    full = lambda shape: pl.BlockSpec(shape, lambda i: (0,) * len(shape))
    return pl.pallas_call(
        _pool_body,
        grid=(nb,),
        in_specs=[pl.BlockSpec((BN, H), lambda i: (i, 0)),
                  full((H, 1)), full((H, H)), full((1, H)),
                  full((H, 32)), full((1, 32)), full((32, 1)), full((1, 1)),
                  full((H, H)), full((1, H)), full((H, A)), full((1, A))],
        out_specs=full((1, A)),
        out_shape=jax.ShapeDtypeStruct((1, A), _f32),
        scratch_shapes=[pltpu.VMEM((1, 1), _f32),
                        pltpu.VMEM((1, 1), _f32),
                        pltpu.VMEM((1, H), _f32)],
    )(x, wp, Wp, bp, Wv1, bv1, Wv2, bv2, Wa1, ba1, Wa2, ba2)


def kernel(node_features, edge_index, edge_features, edge_types,
           W_node, b_node, g_node, be_node,
           W_edge, b_edge, g_edge, be_edge,
           et_emb, Wl, We, a_src, a_dst, a_e,
           w_pool, W_pool, b_pool,
           W_v1, b_v1, W_v2, b_v2, W_a1, b_a1, W_a2, b_a2):
    Wlr = Wl.reshape(L, H, HEADS, DH)
    u_src = jnp.einsum("lkhd,lhd->lkh", Wlr, a_src)
    u_dst = jnp.einsum("lkhd,lhd->lkh", Wlr, a_dst)
    Wer = We.reshape(L, H, HEADS, DH)
    v_e = jnp.einsum("lkhd,lhd->lkh", Wer, a_e)
    v_e_all = jnp.transpose(v_e, (1, 0, 2)).reshape(H, L * 8)

    src = edge_index[0]
    dst = edge_index[1]
    order = jnp.argsort(dst)
    srcp = src[order]
    dstp = dst[order]
    efp = edge_features[order]
    etp = edge_types[order]
    srcp_pad = jnp.concatenate([srcp, jnp.zeros((PAD,), jnp.int32)])
    dstp_pad = jnp.concatenate([dstp, jnp.full((PAD,), NR * RW, jnp.int32)])
    btarg = jnp.arange(0, NR * RW + 1, RW, dtype=jnp.int32)
    bounds = jnp.searchsorted(dstp, btarg).astype(jnp.int32)
    sb = bounds[:NR] & ~15
    nc = (bounds[1:] - sb + C - 1) // C
    sb40 = jnp.zeros((40,), jnp.int32).at[:NR].set(sb)
    nc40 = jnp.zeros((40,), jnp.int32).at[:NR].set(nc)

    et2 = etp.reshape(E, 1)
    r1 = lambda v: v.reshape(1, -1)

    x, h0, h1, asrc_t, adst_t = _node_prep(
        node_features, W_node, r1(b_node), r1(g_node), r1(be_node),
        Wl[0], u_src[0], u_dst[0])

    ae = _edge_prep(efp, et2, W_edge, r1(b_edge), r1(g_edge),
                    r1(be_edge), et_emb, v_e_all)
    zpad = jnp.zeros((PAD // 2, 16), _f32)
    ae = [jnp.concatenate([a, zpad]) for a in ae]

    zacc = jnp.zeros((RW + 4, 36), _f32)

    tpad = jnp.zeros((NR * RW + 8 - N, 8), _f32)
    for l in range(L):
        adst_p = jnp.concatenate([adst_t, tpad])
        a0, a1 = _sc_edge_pass(
            srcp_pad, dstp_pad, asrc_t, adst_p, ae[l], h0, h1, zacc,
            sb40, nc40)
        if l < L - 1:
            x, h0, h1, asrc_t, adst_t = _combine(
                a0, a1, x, Wl[l + 1], u_src[l + 1], u_dst[l + 1])
        else:
            x = _combine(a0, a1, x)

    return _pool_head(
        x, w_pool.reshape(H, 1), W_pool, r1(b_pool),
        W_v1, r1(b_v1), W_v2, r1(b_v2), W_a1, r1(b_a1), W_a2, r1(b_a2))
